# baseline (speedup 1.0000x reference)
import functools

import jax
import jax.numpy as jnp
from jax import lax
from jax.experimental import pallas as pl
from jax.experimental.pallas import tpu as pltpu
from jax.experimental.pallas import tpu_sc as plsc

N = 10000
D_IN = 128
H = 256
HH = H // 2
NG = 64
NC = 2
NS = 16
CH = 128
BLK = 16
ROWS_PT = 640
NPAD = NS * ROWS_PT
R = 1024
GRID = NPAD // R
EPS = 1e-5

_f32 = jnp.float32
_MESH = dict(core_axis_name="c", subcore_axis_name="s", num_cores=NC,
             num_subcores=NS)


def _deg_body(nblk, dst_hbm, cnt0_hbm, cnt1_hbm, cnt_sp, didx, buf, sem):
    c = lax.axis_index("c")
    s = lax.axis_index("s")

    def _fill(val):
        def body(i, _):
            buf[i, :] = jnp.full((16,), val, _f32)
            return 0
        lax.fori_loop(0, CH, body, 0)

    _fill(0.0)
    for k in range(ROWS_PT // CH):
        pltpu.sync_copy(buf, cnt_sp.at[pl.ds(s * ROWS_PT + k * CH, CH)])
    plsc.subcore_barrier()

    _fill(1.0)
    nhalf = (nblk + 1) // 2
    lo = jnp.where(c == 0, 0, nhalf)
    hi = jnp.where(c == 0, nhalf, nblk)

    def blk_body(b, _):
        pltpu.sync_copy(dst_hbm.at[s, b], didx)
        ds = [pltpu.async_copy(buf, cnt_sp.at[didx.at[k]], sem, add=True)
              for k in range(BLK)]
        for d in ds:
            d.wait()
        return 0
    lax.fori_loop(lo, hi, blk_body, 0)
    plsc.subcore_barrier()

    def _copy_out(out_hbm):
        for k in range(ROWS_PT // CH):
            rs = s * ROWS_PT + k * CH
            pltpu.sync_copy(cnt_sp.at[pl.ds(rs, CH)], buf)
            pltpu.sync_copy(buf, out_hbm.at[pl.ds(rs, CH)])

    @pl.when(c == 0)
    def _():
        _copy_out(cnt0_hbm)

    @pl.when(c == 1)
    def _():
        _copy_out(cnt1_hbm)


def _agg_body(nblk, ylo_hbm, yhi_hbm, src_hbm, dst_hbm, olo_hbm, ohi_hbm,
              acc_sp, sidx, didx, rows, sem, sem2):
    c = lax.axis_index("c")
    s = lax.axis_index("s")

    def zbody(i, _):
        for k in range(8):
            rows[0, i, pl.ds(k * 16, 16)] = jnp.zeros((16,), _f32)
        return 0
    lax.fori_loop(0, CH, zbody, 0)
    for k in range(ROWS_PT // CH):
        pltpu.sync_copy(rows.at[0], acc_sp.at[pl.ds(s * ROWS_PT + k * CH, CH)])
    plsc.subcore_barrier()

    nchunks = nblk * BLK

    def _run(table_hbm, out_hbm):
        def edge_body(j, _):
            p = lax.rem(j, 2)
            pltpu.sync_copy(src_hbm.at[s, j], sidx)
            d = pltpu.async_copy(
                table_hbm.at[sidx.at[0]], rows.at[p], sem)
            pltpu.sync_copy(dst_hbm.at[s, j], didx.at[p])

            @pl.when(j > 0)
            def _():
                pltpu.async_copy(rows.at[1 - p],
                                 acc_sp.at[didx.at[1 - p, 0]],
                                 sem2, add=True).wait()
            d.wait()
            return 0
        lax.fori_loop(0, nchunks, edge_body, 0)
        p_last = (nchunks - 1) % 2
        pltpu.async_copy(rows.at[p_last],
                         acc_sp.at[didx.at[p_last, 0]],
                         sem2, add=True).wait()
        plsc.subcore_barrier()
        for k in range(ROWS_PT // CH):
            rs = s * ROWS_PT + k * CH
            pltpu.sync_copy(acc_sp.at[pl.ds(rs, CH)], rows.at[0])
            pltpu.sync_copy(rows.at[0], out_hbm.at[pl.ds(rs, CH)])

    @pl.when(c == 0)
    def _():
        _run(ylo_hbm, olo_hbm)

    @pl.when(c == 1)
    def _():
        _run(yhi_hbm, ohi_hbm)


def _sc_calls(nblk):
    mesh = plsc.VectorSubcoreMesh(**_MESH)
    deg = pl.kernel(
        functools.partial(_deg_body, nblk),
        out_type=(jax.ShapeDtypeStruct((NPAD, 16), _f32),
                  jax.ShapeDtypeStruct((NPAD, 16), _f32)),
        mesh=mesh,
        scratch_types=[
            pltpu.VMEM_SHARED((NPAD, 16), _f32),
            pltpu.VMEM((BLK, CH), jnp.int32),
            pltpu.VMEM((CH, 16), _f32),
            pltpu.SemaphoreType.DMA,
        ],
    )
    agg = pl.kernel(
        functools.partial(_agg_body, nblk),
        out_type=(jax.ShapeDtypeStruct((NPAD, HH), _f32),
                  jax.ShapeDtypeStruct((NPAD, HH), _f32)),
        mesh=mesh,
        scratch_types=[
            pltpu.VMEM_SHARED((NPAD, HH), _f32),
            pltpu.VMEM((1, CH), jnp.int32),
            pltpu.VMEM((2, 1, CH), jnp.int32),
            pltpu.VMEM((2, CH, HH), _f32),
            pltpu.SemaphoreType.DMA,
            pltpu.SemaphoreType.DMA,
        ],
    )
    return deg, agg


def _dis(c0_ref, c1_ref):
    deg = c0_ref[...][:, :1] + c1_ref[...][:, :1] + 1.0
    return lax.rsqrt(deg)


def _y1_body(x_ref, w_ref, c0_ref, c1_ref, olo_ref, ohi_ref):
    y = jnp.dot(x_ref[...], w_ref[...]) * _dis(c0_ref, c1_ref)
    olo_ref[...] = y[:, :HH]
    ohi_ref[...] = y[:, HH:]


def _z(al_ref, ah_ref, yl_ref, yh_ref, c0_ref, c1_ref, b_ref):
    agg = jnp.concatenate([al_ref[...], ah_ref[...]], axis=1)
    y = jnp.concatenate([yl_ref[...], yh_ref[...]], axis=1)
    return _dis(c0_ref, c1_ref) * (agg + y) + b_ref[...]


def _valid_mask(pid):
    rows = pid * R + lax.broadcasted_iota(jnp.int32, (R, 1), 0)
    return (rows < N).astype(_f32)


def _stats_body(al_ref, ah_ref, yl_ref, yh_ref, c0_ref, c1_ref, b_ref,
                st_ref):
    pid = pl.program_id(0)

    @pl.when(pid == 0)
    def _():
        st_ref[...] = jnp.zeros_like(st_ref)

    zm = _z(al_ref, ah_ref, yl_ref, yh_ref, c0_ref, c1_ref,
            b_ref) * _valid_mask(pid)
    st_ref[0:1, :] += jnp.sum(zm, axis=0, keepdims=True)
    st_ref[1:2, :] += jnp.sum(zm * zm, axis=0, keepdims=True)


def _bn_relu(z, st_ref, g_ref, be_ref):
    mu = st_ref[0:1, :] * (1.0 / N)
    var = st_ref[1:2, :] * (1.0 / N) - mu * mu
    inv = lax.rsqrt(var + EPS)
    return jnp.maximum((z - mu) * inv * g_ref[...] + be_ref[...], 0.0)


def _mid_body(al_ref, ah_ref, yl_ref, yh_ref, c0_ref, c1_ref, st_ref, b_ref,
              g_ref, be_ref, w2_ref, olo_ref, ohi_ref):
    z = _z(al_ref, ah_ref, yl_ref, yh_ref, c0_ref, c1_ref, b_ref)
    h = _bn_relu(z, st_ref, g_ref, be_ref)
    y2 = jnp.dot(h, w2_ref[...]) * _dis(c0_ref, c1_ref)
    olo_ref[...] = y2[:, :HH]
    ohi_ref[...] = y2[:, HH:]


def _final_body(al_ref, ah_ref, yl_ref, yh_ref, c0_ref, c1_ref, st_ref,
                b_ref, g_ref, be_ref, bat_ref, wl1_ref, bl1_ref, wl2_ref,
                bl2_ref, out_ref, psum, pcnt):
    pid = pl.program_id(0)

    @pl.when(pid == 0)
    def _():
        psum[...] = jnp.zeros_like(psum)
        pcnt[...] = jnp.zeros_like(pcnt)

    z = _z(al_ref, ah_ref, yl_ref, yh_ref, c0_ref, c1_ref, b_ref)
    h = _bn_relu(z, st_ref, g_ref, be_ref)
    gids = lax.broadcasted_iota(jnp.int32, (R, NG), 1).astype(_f32)
    oh = (bat_ref[...][:, :1] == gids).astype(_f32) * _valid_mask(pid)
    psum[...] += lax.dot_general(oh, h, (((0,), (0,)), ((), ())))
    pcnt[...] += jnp.broadcast_to(jnp.sum(oh, axis=0)[:, None], (NG, 128))

    @pl.when(pid == GRID - 1)
    def _():
        pooled = psum[...] / jnp.maximum(pcnt[...][:, :1], 1.0)
        hh = jnp.maximum(jnp.dot(pooled, wl1_ref[...]) + bl1_ref[...], 0.0)
        out_ref[...] = jnp.dot(hh, wl2_ref[...]) + bl2_ref[...]


def _row_spec(w):
    return pl.BlockSpec((R, w), lambda i: (i, 0))


def _full_spec(shape):
    nd = len(shape)
    return pl.BlockSpec(shape, lambda i: (0,) * nd)


def kernel(x, edge_index, batch, W1, b1, g1, be1, W2, b2, g2, be2,
           Wl1, bl1, Wl2, bl2):
    n_labels = Wl2.shape[1]
    E = edge_index.shape[1]
    nblk = (E + NS * CH * BLK - 1) // (NS * CH * BLK)
    epad = NS * nblk * BLK * CH

    pad_src = jnp.full((epad - E,), N, dtype=jnp.int32)
    pad_dst = N + jnp.arange(epad - E, dtype=jnp.int32) % (NPAD - N)
    srcp = jnp.concatenate([edge_index[0].astype(jnp.int32), pad_src]
                           ).reshape(NS, nblk * BLK, 1, CH)
    dstp = jnp.concatenate([edge_index[1].astype(jnp.int32), pad_dst]
                           ).reshape(NS, nblk * BLK, 1, CH)
    dstb = dstp.reshape(NS, nblk, BLK, CH)
    xp = jnp.concatenate([x, jnp.zeros((NPAD - N, D_IN), _f32)])
    batf = jnp.concatenate([batch.astype(_f32), jnp.full((NPAD - N,), NG, _f32)])
    bat8 = jnp.broadcast_to(batf[:, None], (NPAD, 8))

    b1r, g1r, be1r = b1.reshape(1, H), g1.reshape(1, H), be1.reshape(1, H)
    b2r, g2r, be2r = b2.reshape(1, H), g2.reshape(1, H), be2.reshape(1, H)
    bl1r = bl1.reshape(1, HH)
    wl2p = jnp.zeros((HH, 128), _f32).at[:, :n_labels].set(Wl2)
    bl2p = jnp.zeros((1, 128), _f32).at[0, :n_labels].set(bl2)

    deg_call, agg_call = _sc_calls(nblk)

    cnt0, cnt1 = deg_call(dstb)

    y1lo, y1hi = pl.pallas_call(
        _y1_body,
        grid=(GRID,),
        in_specs=[_row_spec(D_IN), _full_spec((D_IN, H)), _row_spec(16),
                  _row_spec(16)],
        out_specs=[_row_spec(HH), _row_spec(HH)],
        out_shape=[jax.ShapeDtypeStruct((NPAD, HH), _f32)] * 2,
    )(xp, W1, cnt0, cnt1)

    a1lo, a1hi = agg_call(y1lo, y1hi, srcp, dstp)

    stats_call = pl.pallas_call(
        _stats_body,
        grid=(GRID,),
        in_specs=[_row_spec(HH), _row_spec(HH), _row_spec(HH), _row_spec(HH),
                  _row_spec(16), _row_spec(16), _full_spec((1, H))],
        out_specs=pl.BlockSpec((8, H), lambda i: (0, 0)),
        out_shape=jax.ShapeDtypeStruct((8, H), _f32),
    )
    st1 = stats_call(a1lo, a1hi, y1lo, y1hi, cnt0, cnt1, b1r)

    y2lo, y2hi = pl.pallas_call(
        _mid_body,
        grid=(GRID,),
        in_specs=[_row_spec(HH), _row_spec(HH), _row_spec(HH), _row_spec(HH),
                  _row_spec(16), _row_spec(16), _full_spec((8, H)),
                  _full_spec((1, H)), _full_spec((1, H)), _full_spec((1, H)),
                  _full_spec((H, H))],
        out_specs=[_row_spec(HH), _row_spec(HH)],
        out_shape=[jax.ShapeDtypeStruct((NPAD, HH), _f32)] * 2,
    )(a1lo, a1hi, y1lo, y1hi, cnt0, cnt1, st1, b1r, g1r, be1r, W2)

    a2lo, a2hi = agg_call(y2lo, y2hi, srcp, dstp)

    st2 = stats_call(a2lo, a2hi, y2lo, y2hi, cnt0, cnt1, b2r)

    outp = pl.pallas_call(
        _final_body,
        grid=(GRID,),
        in_specs=[_row_spec(HH), _row_spec(HH), _row_spec(HH), _row_spec(HH),
                  _row_spec(16), _row_spec(16), _full_spec((8, H)),
                  _full_spec((1, H)), _full_spec((1, H)), _full_spec((1, H)),
                  _row_spec(8), _full_spec((H, HH)), _full_spec((1, HH)),
                  _full_spec((HH, 128)), _full_spec((1, 128))],
        out_specs=pl.BlockSpec((NG, 128), lambda i: (0, 0)),
        out_shape=jax.ShapeDtypeStruct((NG, 128), _f32),
        scratch_shapes=[pltpu.VMEM((NG, H), _f32), pltpu.VMEM((NG, 128), _f32)],
    )(a2lo, a2hi, y2lo, y2hi, cnt0, cnt1, st2, b2r, g2r, be2r, bat8,
      Wl1, bl1r, wl2p, bl2p)

    return outp[:, :n_labels]


# speedup vs baseline: 1.7655x; 1.7655x over previous
import functools

import jax
import jax.numpy as jnp
from jax import lax
from jax.experimental import pallas as pl
from jax.experimental.pallas import tpu as pltpu
from jax.experimental.pallas import tpu_sc as plsc

N = 10000
D_IN = 128
H = 256
HH = H // 2
NG = 64
NC = 2
NS = 16
CH = 128
BLK = 16
ROWS_PT = 640
NPAD = NS * ROWS_PT
R = 1024
GRID = NPAD // R
EPS = 1e-5

_f32 = jnp.float32
_MESH = dict(core_axis_name="c", subcore_axis_name="s", num_cores=NC,
             num_subcores=NS)


def _deg_body(nchunks, dst_hbm, cnt0_hbm, cnt1_hbm, cnt_sp, dstv, buf, sem):
    c = lax.axis_index("c")
    s = lax.axis_index("s")

    def _fill(val):
        def body(i, _):
            buf[i, :] = jnp.full((16,), val, _f32)
            return 0
        lax.fori_loop(0, CH, body, 0)

    _fill(0.0)
    for k in range(ROWS_PT // CH):
        pltpu.sync_copy(buf, cnt_sp.at[pl.ds(s * ROWS_PT + k * CH, CH)])
    plsc.subcore_barrier()

    _fill(1.0)
    nhalf = (nchunks + 1) // 2
    lo = jnp.where(c == 0, 0, nhalf)
    hi = jnp.where(c == 0, nhalf, nchunks)

    def edge_body(j, _):
        pltpu.sync_copy(dst_hbm.at[s, j], dstv)
        pltpu.sync_copy(buf, cnt_sp.at[dstv.at[0]], add=True)
        return 0
    lax.fori_loop(lo, hi, edge_body, 0)
    plsc.subcore_barrier()

    def _copy_out(out_hbm):
        for k in range(ROWS_PT // CH):
            rs = s * ROWS_PT + k * CH
            pltpu.sync_copy(cnt_sp.at[pl.ds(rs, CH)], buf)
            pltpu.sync_copy(buf, out_hbm.at[pl.ds(rs, CH)])

    @pl.when(c == 0)
    def _():
        _copy_out(cnt0_hbm)

    @pl.when(c == 1)
    def _():
        _copy_out(cnt1_hbm)


def _agg_body(nchunks, ylo_hbm, yhi_hbm, src_hbm, dst_hbm, olo_hbm, ohi_hbm,
              acc_sp, sidx, didx, rows, sem, sem2):
    c = lax.axis_index("c")
    s = lax.axis_index("s")

    def zbody(i, _):
        for k in range(8):
            rows[0, i, pl.ds(k * 16, 16)] = jnp.zeros((16,), _f32)
        return 0
    lax.fori_loop(0, CH, zbody, 0)
    for k in range(ROWS_PT // CH):
        pltpu.sync_copy(rows.at[0], acc_sp.at[pl.ds(s * ROWS_PT + k * CH, CH)])
    plsc.subcore_barrier()

    def _run(table_hbm, out_hbm):
        pltpu.sync_copy(src_hbm.at[s, 0], sidx.at[0])

        def edge_body(j, _):
            p = lax.rem(j, 2)
            d = pltpu.async_copy(
                table_hbm.at[sidx.at[p, 0]], rows.at[p], sem)
            pltpu.sync_copy(dst_hbm.at[s, j], didx.at[p])

            @pl.when(j + 1 < nchunks)
            def _():
                pltpu.sync_copy(src_hbm.at[s, j + 1], sidx.at[1 - p])

            @pl.when(j > 0)
            def _():
                pltpu.async_copy(rows.at[1 - p],
                                 acc_sp.at[didx.at[1 - p, 0]],
                                 sem2, add=True).wait()
            d.wait()
            return 0
        lax.fori_loop(0, nchunks, edge_body, 0)
        p_last = (nchunks - 1) % 2
        pltpu.async_copy(rows.at[p_last],
                         acc_sp.at[didx.at[p_last, 0]],
                         sem2, add=True).wait()
        plsc.subcore_barrier()
        for k in range(ROWS_PT // CH):
            rs = s * ROWS_PT + k * CH
            pltpu.sync_copy(acc_sp.at[pl.ds(rs, CH)], rows.at[0])
            pltpu.sync_copy(rows.at[0], out_hbm.at[pl.ds(rs, CH)])

    @pl.when(c == 0)
    def _():
        _run(ylo_hbm, olo_hbm)

    @pl.when(c == 1)
    def _():
        _run(yhi_hbm, ohi_hbm)


def _sc_calls(nchunks):
    mesh = plsc.VectorSubcoreMesh(**_MESH)
    deg = pl.kernel(
        functools.partial(_deg_body, nchunks),
        out_type=(jax.ShapeDtypeStruct((NPAD, 16), _f32),
                  jax.ShapeDtypeStruct((NPAD, 16), _f32)),
        mesh=mesh,
        scratch_types=[
            pltpu.VMEM_SHARED((NPAD, 16), _f32),
            pltpu.VMEM((1, CH), jnp.int32),
            pltpu.VMEM((CH, 16), _f32),
            pltpu.SemaphoreType.DMA,
        ],
    )
    agg = pl.kernel(
        functools.partial(_agg_body, nchunks),
        out_type=(jax.ShapeDtypeStruct((NPAD, HH), _f32),
                  jax.ShapeDtypeStruct((NPAD, HH), _f32)),
        mesh=mesh,
        scratch_types=[
            pltpu.VMEM_SHARED((NPAD, HH), _f32),
            pltpu.VMEM((2, 1, CH), jnp.int32),
            pltpu.VMEM((2, 1, CH), jnp.int32),
            pltpu.VMEM((2, CH, HH), _f32),
            pltpu.SemaphoreType.DMA,
            pltpu.SemaphoreType.DMA,
        ],
    )
    return deg, agg


def _dis(c0_ref, c1_ref):
    deg = c0_ref[...][:, :1] + c1_ref[...][:, :1] + 1.0
    return lax.rsqrt(deg)


def _y1_body(x_ref, w_ref, c0_ref, c1_ref, olo_ref, ohi_ref):
    y = jnp.dot(x_ref[...], w_ref[...]) * _dis(c0_ref, c1_ref)
    olo_ref[...] = y[:, :HH]
    ohi_ref[...] = y[:, HH:]


def _z(al_ref, ah_ref, yl_ref, yh_ref, c0_ref, c1_ref, b_ref):
    agg = jnp.concatenate([al_ref[...], ah_ref[...]], axis=1)
    y = jnp.concatenate([yl_ref[...], yh_ref[...]], axis=1)
    return _dis(c0_ref, c1_ref) * (agg + y) + b_ref[...]


def _valid_mask(pid):
    rows = pid * R + lax.broadcasted_iota(jnp.int32, (R, 1), 0)
    return (rows < N).astype(_f32)


def _stats_body(al_ref, ah_ref, yl_ref, yh_ref, c0_ref, c1_ref, b_ref,
                st_ref):
    pid = pl.program_id(0)

    @pl.when(pid == 0)
    def _():
        st_ref[...] = jnp.zeros_like(st_ref)

    zm = _z(al_ref, ah_ref, yl_ref, yh_ref, c0_ref, c1_ref,
            b_ref) * _valid_mask(pid)
    st_ref[0:1, :] += jnp.sum(zm, axis=0, keepdims=True)
    st_ref[1:2, :] += jnp.sum(zm * zm, axis=0, keepdims=True)


def _bn_relu(z, st_ref, g_ref, be_ref):
    mu = st_ref[0:1, :] * (1.0 / N)
    var = st_ref[1:2, :] * (1.0 / N) - mu * mu
    inv = lax.rsqrt(var + EPS)
    return jnp.maximum((z - mu) * inv * g_ref[...] + be_ref[...], 0.0)


def _mid_body(al_ref, ah_ref, yl_ref, yh_ref, c0_ref, c1_ref, st_ref, b_ref,
              g_ref, be_ref, w2_ref, olo_ref, ohi_ref):
    z = _z(al_ref, ah_ref, yl_ref, yh_ref, c0_ref, c1_ref, b_ref)
    h = _bn_relu(z, st_ref, g_ref, be_ref)
    y2 = jnp.dot(h, w2_ref[...]) * _dis(c0_ref, c1_ref)
    olo_ref[...] = y2[:, :HH]
    ohi_ref[...] = y2[:, HH:]


def _final_body(al_ref, ah_ref, yl_ref, yh_ref, c0_ref, c1_ref, st_ref,
                b_ref, g_ref, be_ref, bat_ref, wl1_ref, bl1_ref, wl2_ref,
                bl2_ref, out_ref, psum, pcnt):
    pid = pl.program_id(0)

    @pl.when(pid == 0)
    def _():
        psum[...] = jnp.zeros_like(psum)
        pcnt[...] = jnp.zeros_like(pcnt)

    z = _z(al_ref, ah_ref, yl_ref, yh_ref, c0_ref, c1_ref, b_ref)
    h = _bn_relu(z, st_ref, g_ref, be_ref)
    gids = lax.broadcasted_iota(jnp.int32, (R, NG), 1).astype(_f32)
    oh = (bat_ref[...][:, :1] == gids).astype(_f32) * _valid_mask(pid)
    psum[...] += lax.dot_general(oh, h, (((0,), (0,)), ((), ())))
    pcnt[...] += jnp.broadcast_to(jnp.sum(oh, axis=0)[:, None], (NG, 128))

    @pl.when(pid == GRID - 1)
    def _():
        pooled = psum[...] / jnp.maximum(pcnt[...][:, :1], 1.0)
        hh = jnp.maximum(jnp.dot(pooled, wl1_ref[...]) + bl1_ref[...], 0.0)
        out_ref[...] = jnp.dot(hh, wl2_ref[...]) + bl2_ref[...]


def _row_spec(w):
    return pl.BlockSpec((R, w), lambda i: (i, 0))


def _full_spec(shape):
    nd = len(shape)
    return pl.BlockSpec(shape, lambda i: (0,) * nd)


def kernel(x, edge_index, batch, W1, b1, g1, be1, W2, b2, g2, be2,
           Wl1, bl1, Wl2, bl2):
    n_labels = Wl2.shape[1]
    E = edge_index.shape[1]
    nchunks = (E + NS * CH - 1) // (NS * CH)
    epad = NS * nchunks * CH

    pad_src = jnp.full((epad - E,), N, dtype=jnp.int32)
    pad_dst = N + jnp.arange(epad - E, dtype=jnp.int32) % (NPAD - N)
    srcp = jnp.concatenate([edge_index[0].astype(jnp.int32), pad_src]
                           ).reshape(NS, nchunks, 1, CH)
    dstp = jnp.concatenate([edge_index[1].astype(jnp.int32), pad_dst]
                           ).reshape(NS, nchunks, 1, CH)
    xp = jnp.concatenate([x, jnp.zeros((NPAD - N, D_IN), _f32)])
    batf = jnp.concatenate([batch.astype(_f32), jnp.full((NPAD - N,), NG, _f32)])
    bat8 = jnp.broadcast_to(batf[:, None], (NPAD, 8))

    b1r, g1r, be1r = b1.reshape(1, H), g1.reshape(1, H), be1.reshape(1, H)
    b2r, g2r, be2r = b2.reshape(1, H), g2.reshape(1, H), be2.reshape(1, H)
    bl1r = bl1.reshape(1, HH)
    wl2p = jnp.zeros((HH, 128), _f32).at[:, :n_labels].set(Wl2)
    bl2p = jnp.zeros((1, 128), _f32).at[0, :n_labels].set(bl2)

    deg_call, agg_call = _sc_calls(nchunks)

    cnt0, cnt1 = deg_call(dstp)

    y1lo, y1hi = pl.pallas_call(
        _y1_body,
        grid=(GRID,),
        in_specs=[_row_spec(D_IN), _full_spec((D_IN, H)), _row_spec(16),
                  _row_spec(16)],
        out_specs=[_row_spec(HH), _row_spec(HH)],
        out_shape=[jax.ShapeDtypeStruct((NPAD, HH), _f32)] * 2,
    )(xp, W1, cnt0, cnt1)

    a1lo, a1hi = agg_call(y1lo, y1hi, srcp, dstp)

    stats_call = pl.pallas_call(
        _stats_body,
        grid=(GRID,),
        in_specs=[_row_spec(HH), _row_spec(HH), _row_spec(HH), _row_spec(HH),
                  _row_spec(16), _row_spec(16), _full_spec((1, H))],
        out_specs=pl.BlockSpec((8, H), lambda i: (0, 0)),
        out_shape=jax.ShapeDtypeStruct((8, H), _f32),
    )
    st1 = stats_call(a1lo, a1hi, y1lo, y1hi, cnt0, cnt1, b1r)

    y2lo, y2hi = pl.pallas_call(
        _mid_body,
        grid=(GRID,),
        in_specs=[_row_spec(HH), _row_spec(HH), _row_spec(HH), _row_spec(HH),
                  _row_spec(16), _row_spec(16), _full_spec((8, H)),
                  _full_spec((1, H)), _full_spec((1, H)), _full_spec((1, H)),
                  _full_spec((H, H))],
        out_specs=[_row_spec(HH), _row_spec(HH)],
        out_shape=[jax.ShapeDtypeStruct((NPAD, HH), _f32)] * 2,
    )(a1lo, a1hi, y1lo, y1hi, cnt0, cnt1, st1, b1r, g1r, be1r, W2)

    a2lo, a2hi = agg_call(y2lo, y2hi, srcp, dstp)

    st2 = stats_call(a2lo, a2hi, y2lo, y2hi, cnt0, cnt1, b2r)

    outp = pl.pallas_call(
        _final_body,
        grid=(GRID,),
        in_specs=[_row_spec(HH), _row_spec(HH), _row_spec(HH), _row_spec(HH),
                  _row_spec(16), _row_spec(16), _full_spec((8, H)),
                  _full_spec((1, H)), _full_spec((1, H)), _full_spec((1, H)),
                  _row_spec(8), _full_spec((H, HH)), _full_spec((1, HH)),
                  _full_spec((HH, 128)), _full_spec((1, 128))],
        out_specs=pl.BlockSpec((NG, 128), lambda i: (0, 0)),
        out_shape=jax.ShapeDtypeStruct((NG, 128), _f32),
        scratch_shapes=[pltpu.VMEM((NG, H), _f32), pltpu.VMEM((NG, 128), _f32)],
    )(a2lo, a2hi, y2lo, y2hi, cnt0, cnt1, st2, b2r, g2r, be2r, bat8,
      Wl1, bl1r, wl2p, bl2p)

    return outp[:, :n_labels]


# speedup vs baseline: 1.7731x; 1.0043x over previous
import functools

import jax
import jax.numpy as jnp
from jax import lax
from jax.experimental import pallas as pl
from jax.experimental.pallas import tpu as pltpu
from jax.experimental.pallas import tpu_sc as plsc

N = 10000
D_IN = 128
H = 256
HH = H // 2
NG = 64
NC = 2
NS = 16
CH = 128
BLK = 16
ROWS_PT = 640
NPAD = NS * ROWS_PT
R = 1024
GRID = NPAD // R
EPS = 1e-5

_f32 = jnp.float32
_MESH = dict(core_axis_name="c", subcore_axis_name="s", num_cores=NC,
             num_subcores=NS)


def _deg_body(nchunks, dst_hbm, cnt0_hbm, cnt1_hbm, cnt_sp, dstv, buf, sem):
    c = lax.axis_index("c")
    s = lax.axis_index("s")

    def _fill(val):
        def body(i, _):
            buf[i, :] = jnp.full((16,), val, _f32)
            return 0
        lax.fori_loop(0, CH, body, 0)

    _fill(0.0)
    for k in range(ROWS_PT // CH):
        pltpu.sync_copy(buf, cnt_sp.at[pl.ds(s * ROWS_PT + k * CH, CH)])
    plsc.subcore_barrier()

    _fill(1.0)
    nhalf = (nchunks + 1) // 2
    lo = jnp.where(c == 0, 0, nhalf)
    hi = jnp.where(c == 0, nhalf, nchunks)

    def edge_body(j, _):
        pltpu.sync_copy(dst_hbm.at[s, j], dstv)
        pltpu.sync_copy(buf, cnt_sp.at[dstv.at[0]], add=True)
        return 0
    lax.fori_loop(lo, hi, edge_body, 0)
    plsc.subcore_barrier()

    def _copy_out(out_hbm):
        for k in range(ROWS_PT // CH):
            rs = s * ROWS_PT + k * CH
            pltpu.sync_copy(cnt_sp.at[pl.ds(rs, CH)], buf)
            pltpu.sync_copy(buf, out_hbm.at[pl.ds(rs, CH)])

    @pl.when(c == 0)
    def _():
        _copy_out(cnt0_hbm)

    @pl.when(c == 1)
    def _():
        _copy_out(cnt1_hbm)


def _agg_body(nchunks, ylo_hbm, yhi_hbm, src_hbm, dst_hbm, olo_hbm, ohi_hbm,
              acc_sp, sidx, didx, rows, sem, sem2):
    c = lax.axis_index("c")
    s = lax.axis_index("s")

    def zbody(i, _):
        for k in range(8):
            rows[0, i, pl.ds(k * 16, 16)] = jnp.zeros((16,), _f32)
        return 0
    lax.fori_loop(0, CH, zbody, 0)
    for k in range(ROWS_PT // CH):
        pltpu.sync_copy(rows.at[0], acc_sp.at[pl.ds(s * ROWS_PT + k * CH, CH)])
    plsc.subcore_barrier()

    def _run(table_hbm, out_hbm):
        pltpu.sync_copy(src_hbm.at[s, 0], sidx.at[0])

        def edge_body(j, _):
            p = lax.rem(j, 2)
            d = pltpu.async_copy(
                table_hbm.at[sidx.at[p, 0]], rows.at[p], sem)
            pltpu.sync_copy(dst_hbm.at[s, j], didx.at[p])

            @pl.when(j + 1 < nchunks)
            def _():
                pltpu.sync_copy(src_hbm.at[s, j + 1], sidx.at[1 - p])

            @pl.when(j > 0)
            def _():
                pltpu.async_copy(rows.at[1 - p],
                                 acc_sp.at[didx.at[1 - p, 0]],
                                 sem2, add=True).wait()
            d.wait()
            return 0
        lax.fori_loop(0, nchunks, edge_body, 0)
        p_last = (nchunks - 1) % 2
        pltpu.async_copy(rows.at[p_last],
                         acc_sp.at[didx.at[p_last, 0]],
                         sem2, add=True).wait()
        plsc.subcore_barrier()
        nf = ROWS_PT // CH
        fl = [None] * nf
        for k in range(nf):
            rs = s * ROWS_PT + k * CH
            if k >= 2:
                fl[k - 2].wait()
            pltpu.sync_copy(acc_sp.at[pl.ds(rs, CH)], rows.at[k % 2])
            fl[k] = pltpu.async_copy(rows.at[k % 2],
                                     out_hbm.at[pl.ds(rs, CH)], sem)
        fl[nf - 2].wait()
        fl[nf - 1].wait()

    @pl.when(c == 0)
    def _():
        _run(ylo_hbm, olo_hbm)

    @pl.when(c == 1)
    def _():
        _run(yhi_hbm, ohi_hbm)


def _sc_calls(nchunks):
    mesh = plsc.VectorSubcoreMesh(**_MESH)
    deg = pl.kernel(
        functools.partial(_deg_body, nchunks),
        out_type=(jax.ShapeDtypeStruct((NPAD, 16), _f32),
                  jax.ShapeDtypeStruct((NPAD, 16), _f32)),
        mesh=mesh,
        scratch_types=[
            pltpu.VMEM_SHARED((NPAD, 16), _f32),
            pltpu.VMEM((1, CH), jnp.int32),
            pltpu.VMEM((CH, 16), _f32),
            pltpu.SemaphoreType.DMA,
        ],
    )
    agg = pl.kernel(
        functools.partial(_agg_body, nchunks),
        out_type=(jax.ShapeDtypeStruct((NPAD, HH), _f32),
                  jax.ShapeDtypeStruct((NPAD, HH), _f32)),
        mesh=mesh,
        scratch_types=[
            pltpu.VMEM_SHARED((NPAD, HH), _f32),
            pltpu.VMEM((2, 1, CH), jnp.int32),
            pltpu.VMEM((2, 1, CH), jnp.int32),
            pltpu.VMEM((2, CH, HH), _f32),
            pltpu.SemaphoreType.DMA,
            pltpu.SemaphoreType.DMA,
        ],
    )
    return deg, agg


def _dis(c0_ref, c1_ref):
    deg = c0_ref[...][:, :1] + c1_ref[...][:, :1] + 1.0
    return lax.rsqrt(deg)


def _y1_body(x_ref, w_ref, c0_ref, c1_ref, olo_ref, ohi_ref):
    y = jnp.dot(x_ref[...], w_ref[...]) * _dis(c0_ref, c1_ref)
    olo_ref[...] = y[:, :HH]
    ohi_ref[...] = y[:, HH:]


def _z(al_ref, ah_ref, yl_ref, yh_ref, c0_ref, c1_ref, b_ref):
    agg = jnp.concatenate([al_ref[...], ah_ref[...]], axis=1)
    y = jnp.concatenate([yl_ref[...], yh_ref[...]], axis=1)
    return _dis(c0_ref, c1_ref) * (agg + y) + b_ref[...]


def _valid_mask(pid):
    rows = pid * R + lax.broadcasted_iota(jnp.int32, (R, 1), 0)
    return (rows < N).astype(_f32)


def _stats_body(al_ref, ah_ref, yl_ref, yh_ref, c0_ref, c1_ref, b_ref,
                st_ref):
    pid = pl.program_id(0)

    @pl.when(pid == 0)
    def _():
        st_ref[...] = jnp.zeros_like(st_ref)

    zm = _z(al_ref, ah_ref, yl_ref, yh_ref, c0_ref, c1_ref,
            b_ref) * _valid_mask(pid)
    st_ref[0:1, :] += jnp.sum(zm, axis=0, keepdims=True)
    st_ref[1:2, :] += jnp.sum(zm * zm, axis=0, keepdims=True)


def _bn_relu(z, st_ref, g_ref, be_ref):
    mu = st_ref[0:1, :] * (1.0 / N)
    var = st_ref[1:2, :] * (1.0 / N) - mu * mu
    inv = lax.rsqrt(var + EPS)
    return jnp.maximum((z - mu) * inv * g_ref[...] + be_ref[...], 0.0)


def _mid_body(al_ref, ah_ref, yl_ref, yh_ref, c0_ref, c1_ref, st_ref, b_ref,
              g_ref, be_ref, w2_ref, olo_ref, ohi_ref):
    z = _z(al_ref, ah_ref, yl_ref, yh_ref, c0_ref, c1_ref, b_ref)
    h = _bn_relu(z, st_ref, g_ref, be_ref)
    y2 = jnp.dot(h, w2_ref[...]) * _dis(c0_ref, c1_ref)
    olo_ref[...] = y2[:, :HH]
    ohi_ref[...] = y2[:, HH:]


def _final_body(al_ref, ah_ref, yl_ref, yh_ref, c0_ref, c1_ref, st_ref,
                b_ref, g_ref, be_ref, bat_ref, wl1_ref, bl1_ref, wl2_ref,
                bl2_ref, out_ref, psum, pcnt):
    pid = pl.program_id(0)

    @pl.when(pid == 0)
    def _():
        psum[...] = jnp.zeros_like(psum)
        pcnt[...] = jnp.zeros_like(pcnt)

    z = _z(al_ref, ah_ref, yl_ref, yh_ref, c0_ref, c1_ref, b_ref)
    h = _bn_relu(z, st_ref, g_ref, be_ref)
    gids = lax.broadcasted_iota(jnp.int32, (R, NG), 1).astype(_f32)
    oh = (bat_ref[...][:, :1] == gids).astype(_f32) * _valid_mask(pid)
    psum[...] += lax.dot_general(oh, h, (((0,), (0,)), ((), ())))
    pcnt[...] += jnp.broadcast_to(jnp.sum(oh, axis=0)[:, None], (NG, 128))

    @pl.when(pid == GRID - 1)
    def _():
        pooled = psum[...] / jnp.maximum(pcnt[...][:, :1], 1.0)
        hh = jnp.maximum(jnp.dot(pooled, wl1_ref[...]) + bl1_ref[...], 0.0)
        out_ref[...] = jnp.dot(hh, wl2_ref[...]) + bl2_ref[...]


def _row_spec(w):
    return pl.BlockSpec((R, w), lambda i: (i, 0))


def _full_spec(shape):
    nd = len(shape)
    return pl.BlockSpec(shape, lambda i: (0,) * nd)


def kernel(x, edge_index, batch, W1, b1, g1, be1, W2, b2, g2, be2,
           Wl1, bl1, Wl2, bl2):
    n_labels = Wl2.shape[1]
    E = edge_index.shape[1]
    nchunks = (E + NS * CH - 1) // (NS * CH)
    epad = NS * nchunks * CH

    pad_src = jnp.full((epad - E,), N, dtype=jnp.int32)
    pad_dst = N + jnp.arange(epad - E, dtype=jnp.int32) % (NPAD - N)
    srcp = jnp.concatenate([edge_index[0].astype(jnp.int32), pad_src]
                           ).reshape(NS, nchunks, 1, CH)
    dstp = jnp.concatenate([edge_index[1].astype(jnp.int32), pad_dst]
                           ).reshape(NS, nchunks, 1, CH)
    xp = jnp.concatenate([x, jnp.zeros((NPAD - N, D_IN), _f32)])
    batf = jnp.concatenate([batch.astype(_f32), jnp.full((NPAD - N,), NG, _f32)])
    bat8 = jnp.broadcast_to(batf[:, None], (NPAD, 8))

    b1r, g1r, be1r = b1.reshape(1, H), g1.reshape(1, H), be1.reshape(1, H)
    b2r, g2r, be2r = b2.reshape(1, H), g2.reshape(1, H), be2.reshape(1, H)
    bl1r = bl1.reshape(1, HH)
    wl2p = jnp.zeros((HH, 128), _f32).at[:, :n_labels].set(Wl2)
    bl2p = jnp.zeros((1, 128), _f32).at[0, :n_labels].set(bl2)

    deg_call, agg_call = _sc_calls(nchunks)

    cnt0, cnt1 = deg_call(dstp)

    y1lo, y1hi = pl.pallas_call(
        _y1_body,
        grid=(GRID,),
        in_specs=[_row_spec(D_IN), _full_spec((D_IN, H)), _row_spec(16),
                  _row_spec(16)],
        out_specs=[_row_spec(HH), _row_spec(HH)],
        out_shape=[jax.ShapeDtypeStruct((NPAD, HH), _f32)] * 2,
    )(xp, W1, cnt0, cnt1)

    a1lo, a1hi = agg_call(y1lo, y1hi, srcp, dstp)

    stats_call = pl.pallas_call(
        _stats_body,
        grid=(GRID,),
        in_specs=[_row_spec(HH), _row_spec(HH), _row_spec(HH), _row_spec(HH),
                  _row_spec(16), _row_spec(16), _full_spec((1, H))],
        out_specs=pl.BlockSpec((8, H), lambda i: (0, 0)),
        out_shape=jax.ShapeDtypeStruct((8, H), _f32),
    )
    st1 = stats_call(a1lo, a1hi, y1lo, y1hi, cnt0, cnt1, b1r)

    y2lo, y2hi = pl.pallas_call(
        _mid_body,
        grid=(GRID,),
        in_specs=[_row_spec(HH), _row_spec(HH), _row_spec(HH), _row_spec(HH),
                  _row_spec(16), _row_spec(16), _full_spec((8, H)),
                  _full_spec((1, H)), _full_spec((1, H)), _full_spec((1, H)),
                  _full_spec((H, H))],
        out_specs=[_row_spec(HH), _row_spec(HH)],
        out_shape=[jax.ShapeDtypeStruct((NPAD, HH), _f32)] * 2,
    )(a1lo, a1hi, y1lo, y1hi, cnt0, cnt1, st1, b1r, g1r, be1r, W2)

    a2lo, a2hi = agg_call(y2lo, y2hi, srcp, dstp)

    st2 = stats_call(a2lo, a2hi, y2lo, y2hi, cnt0, cnt1, b2r)

    outp = pl.pallas_call(
        _final_body,
        grid=(GRID,),
        in_specs=[_row_spec(HH), _row_spec(HH), _row_spec(HH), _row_spec(HH),
                  _row_spec(16), _row_spec(16), _full_spec((8, H)),
                  _full_spec((1, H)), _full_spec((1, H)), _full_spec((1, H)),
                  _row_spec(8), _full_spec((H, HH)), _full_spec((1, HH)),
                  _full_spec((HH, 128)), _full_spec((1, 128))],
        out_specs=pl.BlockSpec((NG, 128), lambda i: (0, 0)),
        out_shape=jax.ShapeDtypeStruct((NG, 128), _f32),
        scratch_shapes=[pltpu.VMEM((NG, H), _f32), pltpu.VMEM((NG, 128), _f32)],
    )(a2lo, a2hi, y2lo, y2hi, cnt0, cnt1, st2, b2r, g2r, be2r, bat8,
      Wl1, bl1r, wl2p, bl2p)

    return outp[:, :n_labels]
